# baseline (speedup 1.0000x reference)
import jax
import jax.numpy as jnp
import numpy as np
from jax.experimental import pallas as pl


def kernel(frames):
    C, T, H, W = frames.shape
    S = T // 4

    idx = np.linspace(0.0, T - 1, S).astype(np.int64)
    slot_tab = np.searchsorted(idx, np.arange(T), side="left")
    closed = (np.arange(T) * (S - 1) + (T - 2)) // (T - 1)
    assert np.array_equal(slot_tab, closed) and np.all(np.diff(idx) > 0)

    LANES = 128
    HW = H * W
    assert HW % LANES == 0
    SUB = HW // LANES
    x = frames.reshape(C, T, SUB, LANES)

    def body(in_ref, fast_ref, slow_ref):
        v = in_ref[...]
        fast_ref[...] = v
        slow_ref[...] = v

    def slow_map(c, t):
        return (c, (t * (S - 1) + (T - 2)) // (T - 1), 0, 0)

    fast, slow = pl.pallas_call(
        body,
        grid=(C, T),
        in_specs=[pl.BlockSpec((1, 1, SUB, LANES), lambda c, t: (c, t, 0, 0))],
        out_specs=[
            pl.BlockSpec((1, 1, SUB, LANES), lambda c, t: (c, t, 0, 0)),
            pl.BlockSpec((1, 1, SUB, LANES), slow_map),
        ],
        out_shape=[
            jax.ShapeDtypeStruct((C, T, SUB, LANES), frames.dtype),
            jax.ShapeDtypeStruct((C, S, SUB, LANES), frames.dtype),
        ],
    )(x)
    return (slow.reshape(C, S, H, W), fast.reshape(C, T, H, W))


# speedup vs baseline: 1.1773x; 1.1773x over previous
import jax
import jax.numpy as jnp
import numpy as np
from jax.experimental import pallas as pl
from jax.experimental.pallas import tpu as pltpu

_GROUP = 8


def kernel(frames):
    C, T, H, W = frames.shape
    S = T // 4

    idx = np.linspace(0.0, T - 1, S).astype(np.int64)

    LANES = 128
    HW = H * W
    assert HW % LANES == 0
    SUB = HW // LANES
    N = C * T
    x = frames.reshape(N, SUB, LANES)

    G = _GROUP
    assert N % G == 0
    NG = N // G

    sel = [[] for _ in range(NG)]
    for c in range(C):
        for j, t in enumerate(idx):
            r = c * T + int(t)
            sel[r // G].append((r % G, c * S + j))

    def body(in_ref, fast_ref, slow_ref, buf, in_sem, fast_sem, slow_sem):
        def in_cp(g, s):
            return pltpu.make_async_copy(
                in_ref.at[pl.ds(g * G, G)], buf.at[s], in_sem.at[s])

        def fast_cp(g, s):
            return pltpu.make_async_copy(
                buf.at[s], fast_ref.at[pl.ds(g * G, G)], fast_sem.at[s])

        def slow_cp(off, dst, s):
            return pltpu.make_async_copy(
                buf.at[s, off], slow_ref.at[dst], slow_sem.at[s])

        in_cp(0, 0).start()
        for g in range(NG):
            s = g % 2
            in_cp(g, s).wait()
            fast_cp(g, s).start()
            for off, dst in sel[g]:
                slow_cp(off, dst, s).start()
            nxt = g + 1
            if nxt < NG:
                s2 = nxt % 2
                if nxt >= 2:
                    fast_cp(nxt - 2, s2).wait()
                    for off, dst in sel[nxt - 2]:
                        slow_cp(off, dst, s2).wait()
                in_cp(nxt, s2).start()
        for g in range(max(NG - 2, 0), NG):
            s = g % 2
            fast_cp(g, s).wait()
            for off, dst in sel[g]:
                slow_cp(off, dst, s).wait()

    fast, slow = pl.pallas_call(
        body,
        in_specs=[pl.BlockSpec(memory_space=pltpu.MemorySpace.HBM)],
        out_specs=[
            pl.BlockSpec(memory_space=pltpu.MemorySpace.HBM),
            pl.BlockSpec(memory_space=pltpu.MemorySpace.HBM),
        ],
        out_shape=[
            jax.ShapeDtypeStruct((N, SUB, LANES), frames.dtype),
            jax.ShapeDtypeStruct((C * S, SUB, LANES), frames.dtype),
        ],
        scratch_shapes=[
            pltpu.VMEM((2, G, SUB, LANES), frames.dtype),
            pltpu.SemaphoreType.DMA((2,)),
            pltpu.SemaphoreType.DMA((2,)),
            pltpu.SemaphoreType.DMA((2,)),
        ],
    )(x)
    return (slow.reshape(C, S, H, W), fast.reshape(C, T, H, W))


# speedup vs baseline: 1.2874x; 1.0935x over previous
import jax
import jax.numpy as jnp
import numpy as np
from jax.experimental import pallas as pl
from jax.experimental.pallas import tpu as pltpu

_GROUP = 8
_SLOTS = 8


def kernel(frames):
    C, T, H, W = frames.shape
    S = T // 4

    idx = np.linspace(0.0, T - 1, S).astype(np.int64)

    LANES = 128
    HW = H * W
    assert HW % LANES == 0
    SUB = HW // LANES
    N = C * T
    x = frames.reshape(N, SUB, LANES)

    G = _GROUP
    assert N % G == 0
    NG = N // G

    sel = [[] for _ in range(NG)]
    for c in range(C):
        for j, t in enumerate(idx):
            r = c * T + int(t)
            sel[r // G].append((r % G, c * S + j))

    def body(in_ref, fast_ref, slow_ref, buf, in_sem, fast_sem, slow_sem):
        def in_cp(g, s):
            return pltpu.make_async_copy(
                in_ref.at[pl.ds(g * G, G)], buf.at[s], in_sem.at[s])

        def fast_cp(g, s):
            return pltpu.make_async_copy(
                buf.at[s], fast_ref.at[pl.ds(g * G, G)], fast_sem.at[s])

        def slow_cp(off, dst, s):
            return pltpu.make_async_copy(
                buf.at[s, off], slow_ref.at[dst], slow_sem.at[s])

        K = min(_SLOTS, NG)
        for g in range(K):
            in_cp(g, g % K).start()
        for g in range(NG):
            s = g % K
            if g >= 1:
                p = g - 1
                nstage = p + K
                if nstage < NG:
                    sp = p % K
                    fast_cp(p, sp).wait()
                    for off, dst in sel[p]:
                        slow_cp(off, dst, sp).wait()
                    in_cp(nstage, sp).start()
            in_cp(g, s).wait()
            fast_cp(g, s).start()
            for off, dst in sel[g]:
                slow_cp(off, dst, s).start()
        for g in range(max(NG - K, 0), NG):
            if g == NG - 1 or g + K >= NG:
                s = g % K
                fast_cp(g, s).wait()
                for off, dst in sel[g]:
                    slow_cp(off, dst, s).wait()

    fast, slow = pl.pallas_call(
        body,
        in_specs=[pl.BlockSpec(memory_space=pltpu.MemorySpace.HBM)],
        out_specs=[
            pl.BlockSpec(memory_space=pltpu.MemorySpace.HBM),
            pl.BlockSpec(memory_space=pltpu.MemorySpace.HBM),
        ],
        out_shape=[
            jax.ShapeDtypeStruct((N, SUB, LANES), frames.dtype),
            jax.ShapeDtypeStruct((C * S, SUB, LANES), frames.dtype),
        ],
        scratch_shapes=[
            pltpu.VMEM((min(_SLOTS, NG), G, SUB, LANES), frames.dtype),
            pltpu.SemaphoreType.DMA((min(_SLOTS, NG),)),
            pltpu.SemaphoreType.DMA((min(_SLOTS, NG),)),
            pltpu.SemaphoreType.DMA((min(_SLOTS, NG),)),
        ],
    )(x)
    return (slow.reshape(C, S, H, W), fast.reshape(C, T, H, W))
